# baseline (speedup 1.0000x reference)
import functools

import jax
import jax.numpy as jnp
from jax import lax
from jax.experimental import pallas as pl
from jax.experimental.pallas import tpu as pltpu
from jax.experimental.pallas import tpu_sc as plsc

D = 1000
DP = 1024
NC, NS = 2, 16
NW = NC * NS
B, T = 1024, 20
TP = 24
B_PER_W = B // NW
NBUF = 4

_mesh = plsc.VectorSubcoreMesh(
    core_axis_name="c", subcore_axis_name="s", num_cores=NC, num_subcores=NS
)


@functools.partial(
    pl.kernel,
    out_type=jax.ShapeDtypeStruct((B, TP, DP), jnp.float32),
    mesh=_mesh,
    scratch_types=[
        pltpu.VMEM((B_PER_W * TP,), jnp.int32),
        pltpu.VMEM((NBUF, 16, DP), jnp.float32),
        pltpu.VMEM((NBUF, 8, DP), jnp.float32),
        [pltpu.SemaphoreType.DMA] * NBUF,
        [pltpu.SemaphoreType.DMA] * NBUF,
    ],
)
def _gather(idx_hbm, table_hbm, out_hbm, idx_v, bufa, bufc, gsems, wsems):
    wid = lax.axis_index("s") * NC + lax.axis_index("c")
    b0 = wid * B_PER_W
    pltpu.sync_copy(idx_hbm.at[pl.ds(b0 * TP, B_PER_W * TP)], idx_v)

    def gathers(j):
        i = j % NBUF
        return [
            pltpu.async_copy(
                table_hbm.at[idx_v.at[pl.ds(j * TP, 16)]],
                bufa.at[i], gsems[i],
            ),
            pltpu.async_copy(
                table_hbm.at[idx_v.at[pl.ds(j * TP + 16, 8)]],
                bufc.at[i], gsems[i],
            ),
        ]

    def writes(j):
        i = j % NBUF
        b = b0 + j
        return [
            pltpu.async_copy(
                bufa.at[i], out_hbm.at[b, pl.ds(0, 16)], wsems[i],
            ),
            pltpu.async_copy(
                bufc.at[i], out_hbm.at[b, pl.ds(16, 8)], wsems[i],
            ),
        ]

    ghandles = [None] * B_PER_W
    whandles = [None] * B_PER_W
    ghandles[0] = gathers(0)
    ghandles[1] = gathers(1)
    for j in range(B_PER_W):
        if 0 <= j - 2 and j + 2 < B_PER_W:
            for h in whandles[j - 2]:
                h.wait()
        if j + 2 < B_PER_W:
            ghandles[j + 2] = gathers(j + 2)
        for h in ghandles[j]:
            h.wait()
        whandles[j] = writes(j)
    for j in range(B_PER_W - NBUF, B_PER_W):
        for h in whandles[j]:
            h.wait()


def kernel(x, embedding_table):
    table_p = jnp.pad(embedding_table, ((0, 0), (0, DP - D)))
    idx = jnp.pad(x.astype(jnp.int32), ((0, 0), (0, TP - T))).reshape(-1)
    out = _gather(idx, table_p)
    return out[:, :T, :D]


# speedup vs baseline: 2.5725x; 2.5725x over previous
import functools

import jax
import jax.numpy as jnp
from jax import lax
from jax.experimental import pallas as pl
from jax.experimental.pallas import tpu as pltpu
from jax.experimental.pallas import tpu_sc as plsc

D = 1000
DP = 1024
NC, NS = 2, 16
NW = NC * NS
B, T = 1024, 20
TP = 24
B_PER_W = B // NW
NBUF = 4

_mesh = plsc.VectorSubcoreMesh(
    core_axis_name="c", subcore_axis_name="s", num_cores=NC, num_subcores=NS
)


@functools.partial(
    pl.kernel,
    out_type=jax.ShapeDtypeStruct((B, TP, DP), jnp.float32),
    mesh=_mesh,
    scratch_types=[
        pltpu.VMEM((B_PER_W * TP,), jnp.int32),
        pltpu.VMEM((NBUF, 16, DP), jnp.float32),
        pltpu.VMEM((NBUF, 8, DP), jnp.float32),
        [pltpu.SemaphoreType.DMA] * NBUF,
        [pltpu.SemaphoreType.DMA] * NBUF,
    ],
)
def _gather(idx_hbm, table_hbm, out_hbm, idx_v, bufa, bufc, gsems, wsems):
    wid = lax.axis_index("s") * NC + lax.axis_index("c")
    b0 = wid * B_PER_W
    pltpu.sync_copy(idx_hbm.at[pl.ds(b0 * TP, B_PER_W * TP)], idx_v)

    def gathers(j):
        i = j % NBUF
        return [
            pltpu.async_copy(
                table_hbm.at[idx_v.at[pl.ds(j * TP, 16)]],
                bufa.at[i], gsems[i],
            ),
            pltpu.async_copy(
                table_hbm.at[idx_v.at[pl.ds(j * TP + 16, 8)]],
                bufc.at[i], gsems[i],
            ),
        ]

    def writes(j):
        i = j % NBUF
        b = b0 + j
        return [
            pltpu.async_copy(
                bufa.at[i], out_hbm.at[b, pl.ds(0, 16)], wsems[i],
            ),
            pltpu.async_copy(
                bufc.at[i], out_hbm.at[b, pl.ds(16, 8)], wsems[i],
            ),
        ]

    ghandles = [None] * B_PER_W
    whandles = [None] * B_PER_W
    ghandles[0] = gathers(0)
    ghandles[1] = gathers(1)
    for j in range(B_PER_W):
        if 0 <= j - 2 and j + 2 < B_PER_W:
            for h in whandles[j - 2]:
                h.wait()
        if j + 2 < B_PER_W:
            ghandles[j + 2] = gathers(j + 2)
        for h in ghandles[j]:
            h.wait()
        whandles[j] = writes(j)
    for j in range(B_PER_W - NBUF, B_PER_W):
        for h in whandles[j]:
            h.wait()


def kernel(x, embedding_table):
    table_p = jnp.pad(embedding_table, ((0, 0), (0, DP - D)))
    xi = x.astype(jnp.int32)
    idx = jnp.concatenate([xi, xi[:, : TP - T]], axis=1).reshape(-1)
    out = _gather(idx, table_p)
    return out[:, :T, :D]


# speedup vs baseline: 2.8119x; 1.0931x over previous
import functools

import jax
import jax.numpy as jnp
from jax import lax
from jax.experimental import pallas as pl
from jax.experimental.pallas import tpu as pltpu
from jax.experimental.pallas import tpu_sc as plsc

D = 1000
DP = 1024
NC, NS = 2, 16
NW = NC * NS
B, T = 1024, 20
TP = 24
B_PER_W = B // NW
NBUF = 6

_mesh = plsc.VectorSubcoreMesh(
    core_axis_name="c", subcore_axis_name="s", num_cores=NC, num_subcores=NS
)


@functools.partial(
    pl.kernel,
    out_type=jax.ShapeDtypeStruct((B, T, DP), jnp.float32),
    mesh=_mesh,
    scratch_types=[
        pltpu.VMEM((B_PER_W * TP,), jnp.int32),
        pltpu.VMEM((NBUF, 16, DP), jnp.float32),
        pltpu.VMEM((NBUF, 4, DP), jnp.float32),
        [pltpu.SemaphoreType.DMA] * NBUF,
        [pltpu.SemaphoreType.DMA] * NBUF,
    ],
)
def _gather(idx_hbm, table_hbm, out_hbm, idx_v, bufa, bufc, gsems, wsems):
    wid = lax.axis_index("s") * NC + lax.axis_index("c")
    b0 = wid * B_PER_W
    pltpu.sync_copy(idx_hbm.at[pl.ds(b0 * TP, B_PER_W * TP)], idx_v)

    def gathers(j):
        i = j % NBUF
        return [
            pltpu.async_copy(
                table_hbm.at[idx_v.at[pl.ds(j * TP, 16)]],
                bufa.at[i], gsems[i],
            ),
            pltpu.async_copy(
                table_hbm.at[idx_v.at[pl.ds(j * TP + 16, 4)]],
                bufc.at[i], gsems[i],
            ),
        ]

    def writes(j):
        i = j % NBUF
        b = b0 + j
        return [
            pltpu.async_copy(
                bufa.at[i], out_hbm.at[b, pl.ds(0, 16)], wsems[i],
            ),
            pltpu.async_copy(
                bufc.at[i], out_hbm.at[b, pl.ds(16, 4)], wsems[i],
            ),
        ]

    ghandles = [None] * B_PER_W
    whandles = [None] * B_PER_W
    for j in range(3):
        ghandles[j] = gathers(j)
    for j in range(B_PER_W):
        if 0 <= j - 3 and j + 3 < B_PER_W:
            for h in whandles[j - 3]:
                h.wait()
        if j + 3 < B_PER_W:
            ghandles[j + 3] = gathers(j + 3)
        for h in ghandles[j]:
            h.wait()
        whandles[j] = writes(j)
    for j in range(B_PER_W - NBUF, B_PER_W):
        for h in whandles[j]:
            h.wait()


def kernel(x, embedding_table):
    table_p = jnp.pad(embedding_table, ((0, 0), (0, DP - D)))
    idx = jnp.pad(x.astype(jnp.int32), ((0, 0), (0, TP - T))).reshape(-1)
    out = _gather(idx, table_p)
    return out[:, :, :D]
